# baseline (speedup 1.0000x reference)
import functools

import jax
import jax.numpy as jnp
from jax import lax
from jax.experimental import pallas as pl
from jax.experimental.pallas import tpu as pltpu
from jax.experimental.pallas import tpu_sc as plsc

B, T, H, V = 1024, 200, 128, 256
NC, NS = 2, 16
NW = NC * NS
BPW = B // NW
BB = 16


def _gather_body(idx_hbm, table_hbm, emb_hbm, idx_v, rows_v, gat_sem):
    wid = lax.axis_index("s") * NC + lax.axis_index("c")
    base = wid * BPW
    pltpu.sync_copy(idx_hbm.at[pl.ds(base, BPW)], idx_v)
    pltpu.async_copy(table_hbm.at[idx_v], rows_v.at[:, 0, :], gat_sem).wait()
    pltpu.sync_copy(rows_v, emb_hbm.at[pl.ds(base, BPW)])


def _concat_body(emb_ref, mel_ref, out_ref):
    out_ref[:, 0:1, :] = emb_ref[...]
    out_ref[:, 1:, :] = mel_ref[...]


@jax.jit
def _run(idx, table, mel):
    mesh = plsc.VectorSubcoreMesh(
        core_axis_name="c", subcore_axis_name="s", num_cores=NC, num_subcores=NS
    )
    emb = pl.kernel(
        _gather_body,
        out_type=jax.ShapeDtypeStruct((B, 1, H), jnp.float32),
        mesh=mesh,
        scratch_types=[
            pltpu.VMEM((BPW,), jnp.int32),
            pltpu.VMEM((BPW, 1, H), jnp.float32),
            pltpu.SemaphoreType.DMA,
        ],
        compiler_params=pltpu.CompilerParams(use_tc_tiling_on_sc=True),
    )(idx, table)

    return pl.pallas_call(
        _concat_body,
        grid=(B // BB,),
        in_specs=[
            pl.BlockSpec((BB, 1, H), lambda i: (i, 0, 0)),
            pl.BlockSpec((BB, T, H), lambda i: (i, 0, 0)),
        ],
        out_specs=pl.BlockSpec((BB, T + 1, H), lambda i: (i, 0, 0)),
        out_shape=jax.ShapeDtypeStruct((B, T + 1, H), jnp.float32),
        compiler_params=pltpu.CompilerParams(
            dimension_semantics=("arbitrary",),
        ),
    )(emb, mel)


def kernel(arranger_id, mel_db, table):
    idx = arranger_id.reshape(B).astype(jnp.int32)
    return _run(idx, table, mel_db)


# speedup vs baseline: 1.7024x; 1.7024x over previous
import jax
import jax.numpy as jnp
from jax import lax
from jax.experimental import pallas as pl
from jax.experimental.pallas import tpu as pltpu
from jax.experimental.pallas import tpu_sc as plsc

B, T, H, V = 1024, 200, 128, 256
NC, NS = 2, 16
NW = NC * NS
BPW = B // NW
TB = 40
NTB = T // TB
NBG = BPW // 8


def _sc_body(idx_hbm, table_hbm, mel_hbm, outt_hbm,
             idx_v, rows_v, bufs, rsems, wsems, gat_sem):
    wid = lax.axis_index("s") * NC + lax.axis_index("c")
    base = wid * BPW

    pltpu.sync_copy(idx_hbm.at[pl.ds(base, BPW)], idx_v)
    pltpu.async_copy(table_hbm.at[idx_v], rows_v.at[0], gat_sem).wait()
    emb_wr = pltpu.async_copy(
        rows_v, outt_hbm.at[pl.ds(0, 1), pl.ds(base, BPW), :], gat_sem
    )

    blocks = [(g, t) for g in range(NBG) for t in range(NTB)]
    nblk = len(blocks)
    rd = [None] * nblk
    wr = [None] * nblk

    def start_read(i, buf, sems):
        g, t = blocks[i]
        return [
            pltpu.async_copy(
                mel_hbm.at[base + g * 8 + bb, pl.ds(t * TB, TB), :],
                buf.at[:, bb, :],
                sems,
            )
            for bb in range(8)
        ]

    rd[0] = start_read(0, bufs[0], rsems[0])
    rd[1] = start_read(1, bufs[1], rsems[1])
    for i in range(nblk):
        b = i % 2
        for d in rd[i]:
            d.wait()
        g, t = blocks[i]
        wr[i] = pltpu.async_copy(
            bufs[b],
            outt_hbm.at[pl.ds(1 + t * TB, TB), pl.ds(base + g * 8, 8), :],
            wsems[b],
        )
        if i + 2 < nblk:
            wr[i].wait()
            rd[i + 2] = start_read(i + 2, bufs[b], rsems[b])
    wr[nblk - 2].wait()
    wr[nblk - 1].wait()
    emb_wr.wait()


@jax.jit
def _run(idx, table, mel):
    mesh = plsc.VectorSubcoreMesh(
        core_axis_name="c", subcore_axis_name="s", num_cores=NC, num_subcores=NS
    )
    outt = pl.kernel(
        _sc_body,
        out_type=jax.ShapeDtypeStruct((T + 1, B, H), jnp.float32),
        mesh=mesh,
        scratch_types=[
            pltpu.VMEM((BPW,), jnp.int32),
            pltpu.VMEM((1, BPW, H), jnp.float32),
            [pltpu.VMEM((TB, 8, H), jnp.float32)] * 2,
            [pltpu.SemaphoreType.DMA] * 2,
            [pltpu.SemaphoreType.DMA] * 2,
            pltpu.SemaphoreType.DMA,
        ],
        compiler_params=pltpu.CompilerParams(use_tc_tiling_on_sc=True),
    )(idx, table, mel)
    return outt.transpose(1, 0, 2)


def kernel(arranger_id, mel_db, table):
    idx = arranger_id.reshape(B).astype(jnp.int32)
    return _run(idx, table, mel_db)
